# Initial kernel scaffold; baseline (speedup 1.0000x reference)
import jax
import jax.numpy as jnp
from jax.experimental import pallas as pl


def kernel(graph_x, graph_edge_index, graph_batch, x, drugEdges, seEdges, drugNodes, seNodes, proteinNodes, proteinWeight, nDrug, emb, Wg1, as1, ad1, bg1, Wg2, as2, ad2, bg2, Wg3, as3, ad3, bg3, pw1, pw2, pw3, lin1_W, lin1_b, Wd1, bd1, Wd2, bd2):
    raise NotImplementedError("write your pallas kernel here")


# speedup vs baseline: 1.1548x; 1.1548x over previous
import functools

import jax
import jax.numpy as jnp
from jax import lax
from jax.experimental import pallas as pl
from jax.experimental.pallas import tpu as pltpu
from jax.experimental.pallas import tpu_sc as plsc

_D = 128
_NC, _NS = 2, 16
_NW = _NC * _NS


def _make_sc_gather(V, D, B):
    bpw = B // _NW
    mesh = plsc.VectorSubcoreMesh(core_axis_name="c", subcore_axis_name="s")

    @functools.partial(
        pl.kernel, mesh=mesh,
        out_type=jax.ShapeDtypeStruct((B, D), jnp.float32),
        scratch_types=[
            pltpu.VMEM((bpw,), jnp.int32),
            pltpu.VMEM((bpw, D), jnp.float32),
            pltpu.SemaphoreType.DMA,
        ],
    )
    def k(table_hbm, idx_hbm, out_hbm, idx_v, rows_v, sem):
        wid = lax.axis_index("s") * _NC + lax.axis_index("c")
        base = wid * bpw
        pltpu.sync_copy(idx_hbm.at[pl.ds(base, bpw)], idx_v)
        pltpu.async_copy(table_hbm.at[idx_v], rows_v, sem).wait()
        pltpu.sync_copy(rows_v, out_hbm.at[pl.ds(base, bpw)])

    return k


def _gather_rows(table, idx):
    B = idx.shape[0]
    Bp = -(-B // (8 * _NW)) * (8 * _NW)
    idxp = idx.astype(jnp.int32)
    if Bp != B:
        idxp = jnp.pad(idxp, (0, Bp - B))
    out = _make_sc_gather(table.shape[0], table.shape[1], Bp)(table, idxp)
    return out[:B]


def _gat_dense(xa, W, a_s, a_d):
    n = xa.shape[0]

    def body(x_ref, w_ref, as_ref, ad_ref, h_ref, es_ref, ed_ref):
        h = jnp.dot(x_ref[...], w_ref[...], preferred_element_type=jnp.float32)
        h_ref[...] = h
        es_ref[...] = jnp.sum(h * as_ref[...], axis=1, keepdims=True)
        ed_ref[...] = jnp.sum(h * ad_ref[...], axis=1, keepdims=True)

    h, es, ed = pl.pallas_call(
        body,
        out_shape=(
            jax.ShapeDtypeStruct((n, _D), jnp.float32),
            jax.ShapeDtypeStruct((n, 1), jnp.float32),
            jax.ShapeDtypeStruct((n, 1), jnp.float32),
        ),
    )(xa, W, a_s.reshape(1, _D), a_d.reshape(1, _D))
    return h, es[:, 0], ed[:, 0]


def _bias_relu_pool(agg, b, w):
    n = agg.shape[0]

    def body(a_ref, b_ref, w_ref, o_ref):
        xf = jnp.maximum(a_ref[...] + b_ref[...], 0.0)
        wv = w_ref[...]
        nrm = jnp.sqrt(jnp.sum(wv * wv))
        score = jnp.sum(xf * wv, axis=1, keepdims=True) / jnp.maximum(nrm, 1e-12)
        o_ref[...] = xf * jnp.tanh(score)

    return pl.pallas_call(
        body, out_shape=jax.ShapeDtypeStruct((n, _D), jnp.float32),
    )(agg, b.reshape(1, _D), w.reshape(1, _D))


def _lin1(x1, x2, x3, W, b):
    n = x1.shape[0]

    def body(a_ref, b2_ref, c_ref, w_ref, bias_ref, o_ref):
        s = a_ref[...] + b2_ref[...] + c_ref[...]
        o_ref[...] = jnp.maximum(
            jnp.dot(s, w_ref[...], preferred_element_type=jnp.float32)
            + bias_ref[...], 0.0)

    return pl.pallas_call(
        body, out_shape=jax.ShapeDtypeStruct((n, _D), jnp.float32),
    )(x1, x2, x3, W, b.reshape(1, _D))


def _mm(x, W):
    n = x.shape[0]

    def body(x_ref, w_ref, o_ref):
        o_ref[...] = jnp.dot(x_ref[...], w_ref[...],
                             preferred_element_type=jnp.float32)

    return pl.pallas_call(
        body, out_shape=jax.ShapeDtypeStruct((n, _D), jnp.float32),
    )(x, W)


def _bias_relu(agg, b):
    n = agg.shape[0]

    def body(a_ref, b_ref, o_ref):
        o_ref[...] = jnp.maximum(a_ref[...] + b_ref[...], 0.0)

    return pl.pallas_call(
        body, out_shape=jax.ShapeDtypeStruct((n, _D), jnp.float32),
    )(agg, b.reshape(1, _D))


def _seg_sum(d, i, n):
    return jax.ops.segment_sum(d, i, num_segments=n)


def _seg_max(d, i, n):
    return jax.ops.segment_max(d, i, num_segments=n)


def _gat_layer(xa, src, dst, W, a_s, a_d, b, pw):
    n = xa.shape[0]
    h, es, ed = _gat_dense(xa, W, a_s, a_d)
    e = jax.nn.leaky_relu(es[src] + ed[dst], 0.2)
    m = _seg_max(e, dst, n)
    m = jnp.where(jnp.isfinite(m), m, 0.0)
    ex = jnp.exp(e - m[dst])
    den = _seg_sum(ex, dst, n)
    coef = ex / jnp.maximum(den[dst], 1e-16)
    agg = _seg_sum(coef[:, None] * h[src], dst, n)
    return _bias_relu_pool(agg, b, pw)


def _gmp_gap(xf, batch, n_graphs):
    cnt = _seg_sum(jnp.ones(xf.shape[0], jnp.float32), batch, n_graphs)
    mx = _seg_max(xf, batch, n_graphs)
    mx = jnp.where((cnt > 0)[:, None], mx, 0.0)
    mean = _seg_sum(xf, batch, n_graphs) / jnp.maximum(cnt, 1.0)[:, None]
    return jnp.concatenate([mx, mean], axis=1)


def kernel(graph_x, graph_edge_index, graph_batch, x, drugEdges, seEdges,
           drugNodes, seNodes, proteinNodes, proteinWeight, nDrug, emb,
           Wg1, as1, ad1, bg1, Wg2, as2, ad2, bg2, Wg3, as3, ad3, bg3,
           pw1, pw2, pw3, lin1_W, lin1_b, Wd1, bd1, Wd2, bd2):
    n = graph_x.shape[0]
    n_graphs = 3000
    loop = jnp.arange(n, dtype=graph_edge_index.dtype)
    src = jnp.concatenate([graph_edge_index[0], loop])
    dst = jnp.concatenate([graph_edge_index[1], loop])

    xa = _gather_rows(emb, graph_x)
    xa = _gat_layer(xa, src, dst, Wg1, as1, ad1, bg1, pw1)
    x1 = _gmp_gap(xa, graph_batch, n_graphs)
    xa = _gat_layer(xa, src, dst, Wg2, as2, ad2, bg2, pw2)
    x2 = _gmp_gap(xa, graph_batch, n_graphs)
    xa = _gat_layer(xa, src, dst, Wg3, as3, ad3, bg3, pw3)
    x3 = _gmp_gap(xa, graph_batch, n_graphs)
    xdp = _lin1(x1, x2, x3, lin1_W, lin1_b)

    nProtein = proteinNodes.shape[0]
    n_se = x.shape[0] - drugNodes.shape[0] - nProtein
    se_idx = lax.dynamic_slice_in_dim(x, nDrug + nProtein, n_se)
    se = _gather_rows(emb, se_idx)
    X = jnp.concatenate([xdp, se], axis=0)

    nd = X.shape[0]
    dloop = jnp.arange(nd, dtype=drugEdges.dtype)
    dsrc = jnp.concatenate([drugEdges[0], dloop])
    ddst = jnp.concatenate([drugEdges[1], dloop])
    deg = _seg_sum(jnp.ones(dsrc.shape[0], jnp.float32), ddst, nd)
    dis = jnp.where(deg > 0, lax.rsqrt(jnp.maximum(deg, 1e-12)), 0.0)
    norm = dis[dsrc] * dis[ddst]

    h1 = _mm(X, Wd1)
    X = _bias_relu(_seg_sum(norm[:, None] * h1[dsrc], ddst, nd), bd1)
    h2 = _mm(X, Wd2)
    X = _bias_relu(_seg_sum(norm[:, None] * h2[dsrc], ddst, nd), bd2)

    return (_gather_rows(X, drugNodes), _gather_rows(X, seNodes), X)


# speedup vs baseline: 1.2183x; 1.0549x over previous
import functools

import jax
import jax.numpy as jnp
from jax import lax
from jax.experimental import pallas as pl
from jax.experimental.pallas import tpu as pltpu
from jax.experimental.pallas import tpu_sc as plsc

_D = 128
_NC, _NS = 2, 16
_NW = _NC * _NS


def _make_sc_gather(V, D, B):
    bpw = B // _NW
    mesh = plsc.VectorSubcoreMesh(core_axis_name="c", subcore_axis_name="s")

    @functools.partial(
        pl.kernel, mesh=mesh,
        out_type=jax.ShapeDtypeStruct((B, D), jnp.float32),
        scratch_types=[
            pltpu.VMEM((bpw,), jnp.int32),
            pltpu.VMEM((bpw, D), jnp.float32),
            pltpu.SemaphoreType.DMA,
        ],
    )
    def k(table_hbm, idx_hbm, out_hbm, idx_v, rows_v, sem):
        wid = lax.axis_index("s") * _NC + lax.axis_index("c")
        base = wid * bpw
        pltpu.sync_copy(idx_hbm.at[pl.ds(base, bpw)], idx_v)
        pltpu.async_copy(table_hbm.at[idx_v], rows_v, sem).wait()
        pltpu.sync_copy(rows_v, out_hbm.at[pl.ds(base, bpw)])

    return k


def _gather_rows(table, idx):
    B = idx.shape[0]
    Bp = -(-B // (8 * _NW)) * (8 * _NW)
    idxp = idx.astype(jnp.int32)
    if Bp != B:
        idxp = jnp.pad(idxp, (0, Bp - B))
    out = _make_sc_gather(table.shape[0], table.shape[1], Bp)(table, idxp)
    return out[:B]


def _gat_dense(xa, W, a_s, a_d):
    n = xa.shape[0]

    def body(x_ref, w_ref, as_ref, ad_ref, h_ref, es_ref, ed_ref):
        h = jnp.dot(x_ref[...], w_ref[...], preferred_element_type=jnp.float32)
        h_ref[...] = h
        es_ref[...] = jnp.sum(h * as_ref[...], axis=1, keepdims=True)
        ed_ref[...] = jnp.sum(h * ad_ref[...], axis=1, keepdims=True)

    h, es, ed = pl.pallas_call(
        body,
        out_shape=(
            jax.ShapeDtypeStruct((n, _D), jnp.float32),
            jax.ShapeDtypeStruct((n, 1), jnp.float32),
            jax.ShapeDtypeStruct((n, 1), jnp.float32),
        ),
    )(xa, W, a_s.reshape(1, _D), a_d.reshape(1, _D))
    return h, es[:, 0], ed[:, 0]


def _bias_relu_pool(agg, b, w):
    n = agg.shape[0]

    def body(a_ref, b_ref, w_ref, o_ref):
        xf = jnp.maximum(a_ref[...] + b_ref[...], 0.0)
        wv = w_ref[...]
        nrm = jnp.sqrt(jnp.sum(wv * wv))
        score = jnp.sum(xf * wv, axis=1, keepdims=True) / jnp.maximum(nrm, 1e-12)
        o_ref[...] = xf * jnp.tanh(score)

    return pl.pallas_call(
        body, out_shape=jax.ShapeDtypeStruct((n, _D), jnp.float32),
    )(agg, b.reshape(1, _D), w.reshape(1, _D))


def _lin1(x1, x2, x3, W, b):
    n = x1.shape[0]

    def body(a_ref, b2_ref, c_ref, w_ref, bias_ref, o_ref):
        s = a_ref[...] + b2_ref[...] + c_ref[...]
        o_ref[...] = jnp.maximum(
            jnp.dot(s, w_ref[...], preferred_element_type=jnp.float32)
            + bias_ref[...], 0.0)

    return pl.pallas_call(
        body, out_shape=jax.ShapeDtypeStruct((n, _D), jnp.float32),
    )(x1, x2, x3, W, b.reshape(1, _D))


def _mm(x, W):
    n = x.shape[0]

    def body(x_ref, w_ref, o_ref):
        o_ref[...] = jnp.dot(x_ref[...], w_ref[...],
                             preferred_element_type=jnp.float32)

    return pl.pallas_call(
        body, out_shape=jax.ShapeDtypeStruct((n, _D), jnp.float32),
    )(x, W)


def _bias_relu(agg, b):
    n = agg.shape[0]

    def body(a_ref, b_ref, o_ref):
        o_ref[...] = jnp.maximum(a_ref[...] + b_ref[...], 0.0)

    return pl.pallas_call(
        body, out_shape=jax.ShapeDtypeStruct((n, _D), jnp.float32),
    )(agg, b.reshape(1, _D))


def _seg_sum(d, i, n, srt=False):
    return jax.ops.segment_sum(d, i, num_segments=n, indices_are_sorted=srt)


def _seg_max(d, i, n, srt=False):
    return jax.ops.segment_max(d, i, num_segments=n, indices_are_sorted=srt)


def _gat_layer(xa, src, dst, W, a_s, a_d, b, pw):
    n = xa.shape[0]
    h, es, ed = _gat_dense(xa, W, a_s, a_d)
    e = jax.nn.leaky_relu(es[src] + ed[dst], 0.2)
    m = _seg_max(e, dst, n, srt=True)
    m = jnp.where(jnp.isfinite(m), m, 0.0)
    ex = jnp.exp(e - m[dst])
    den = _seg_sum(ex, dst, n, srt=True)
    coef = ex / jnp.maximum(den[dst], 1e-16)
    agg = _seg_sum(coef[:, None] * h[src], dst, n, srt=True)
    return _bias_relu_pool(agg, b, pw)


def _gmp_gap(xf, batch, n_graphs):
    cnt = _seg_sum(jnp.ones(xf.shape[0], jnp.float32), batch, n_graphs, srt=True)
    mx = _seg_max(xf, batch, n_graphs, srt=True)
    mx = jnp.where((cnt > 0)[:, None], mx, 0.0)
    mean = _seg_sum(xf, batch, n_graphs, srt=True) / jnp.maximum(cnt, 1.0)[:, None]
    return jnp.concatenate([mx, mean], axis=1)


def kernel(graph_x, graph_edge_index, graph_batch, x, drugEdges, seEdges,
           drugNodes, seNodes, proteinNodes, proteinWeight, nDrug, emb,
           Wg1, as1, ad1, bg1, Wg2, as2, ad2, bg2, Wg3, as3, ad3, bg3,
           pw1, pw2, pw3, lin1_W, lin1_b, Wd1, bd1, Wd2, bd2):
    n = graph_x.shape[0]
    n_graphs = 3000
    loop = jnp.arange(n, dtype=graph_edge_index.dtype)
    src = jnp.concatenate([graph_edge_index[0], loop])
    dst = jnp.concatenate([graph_edge_index[1], loop])
    order = jnp.argsort(dst)
    src, dst = src[order], dst[order]

    xa = _gather_rows(emb, graph_x)
    xa = _gat_layer(xa, src, dst, Wg1, as1, ad1, bg1, pw1)
    x1 = _gmp_gap(xa, graph_batch, n_graphs)
    xa = _gat_layer(xa, src, dst, Wg2, as2, ad2, bg2, pw2)
    x2 = _gmp_gap(xa, graph_batch, n_graphs)
    xa = _gat_layer(xa, src, dst, Wg3, as3, ad3, bg3, pw3)
    x3 = _gmp_gap(xa, graph_batch, n_graphs)
    xdp = _lin1(x1, x2, x3, lin1_W, lin1_b)

    nProtein = proteinNodes.shape[0]
    n_se = x.shape[0] - drugNodes.shape[0] - nProtein
    se_idx = lax.dynamic_slice_in_dim(x, nDrug + nProtein, n_se)
    se = _gather_rows(emb, se_idx)
    X = jnp.concatenate([xdp, se], axis=0)

    nd = X.shape[0]
    dloop = jnp.arange(nd, dtype=drugEdges.dtype)
    dsrc = jnp.concatenate([drugEdges[0], dloop])
    ddst = jnp.concatenate([drugEdges[1], dloop])
    dorder = jnp.argsort(ddst)
    dsrc, ddst = dsrc[dorder], ddst[dorder]
    deg = _seg_sum(jnp.ones(dsrc.shape[0], jnp.float32), ddst, nd, srt=True)
    dis = jnp.where(deg > 0, lax.rsqrt(jnp.maximum(deg, 1e-12)), 0.0)
    norm = dis[dsrc] * dis[ddst]

    h1 = _mm(X, Wd1)
    X = _bias_relu(_seg_sum(norm[:, None] * h1[dsrc], ddst, nd, srt=True), bd1)
    h2 = _mm(X, Wd2)
    X = _bias_relu(_seg_sum(norm[:, None] * h2[dsrc], ddst, nd, srt=True), bd2)

    return (_gather_rows(X, drugNodes), _gather_rows(X, seNodes), X)


# speedup vs baseline: 1.3963x; 1.1461x over previous
import functools

import jax
import jax.numpy as jnp
from jax import lax
from jax.experimental import pallas as pl
from jax.experimental.pallas import tpu as pltpu
from jax.experimental.pallas import tpu_sc as plsc

_D = 128
_NC, _NS = 2, 16
_NW = _NC * _NS


def _make_sc_gather(V, D, B):
    bpw = B // _NW
    mesh = plsc.VectorSubcoreMesh(core_axis_name="c", subcore_axis_name="s")

    @functools.partial(
        pl.kernel, mesh=mesh,
        out_type=jax.ShapeDtypeStruct((B, D), jnp.float32),
        scratch_types=[
            pltpu.VMEM((bpw,), jnp.int32),
            pltpu.VMEM((bpw, D), jnp.float32),
            pltpu.SemaphoreType.DMA,
        ],
    )
    def k(table_hbm, idx_hbm, out_hbm, idx_v, rows_v, sem):
        wid = lax.axis_index("s") * _NC + lax.axis_index("c")
        base = wid * bpw
        pltpu.sync_copy(idx_hbm.at[pl.ds(base, bpw)], idx_v)
        pltpu.async_copy(table_hbm.at[idx_v], rows_v, sem).wait()
        pltpu.sync_copy(rows_v, out_hbm.at[pl.ds(base, bpw)])

    return k


def _gather_rows(table, idx):
    B = idx.shape[0]
    Bp = -(-B // (8 * _NW)) * (8 * _NW)
    idxp = idx.astype(jnp.int32)
    if Bp != B:
        idxp = jnp.pad(idxp, (0, Bp - B))
    out = _make_sc_gather(table.shape[0], table.shape[1], Bp)(table, idxp)
    return out[:B]


def _gat_dense(xa, W, a_s, a_d):
    n = xa.shape[0]

    def body(x_ref, w_ref, as_ref, ad_ref, h_ref, es_ref, ed_ref):
        h = jnp.dot(x_ref[...], w_ref[...], preferred_element_type=jnp.float32)
        h_ref[...] = h
        es_ref[...] = jnp.sum(h * as_ref[...], axis=1, keepdims=True)
        ed_ref[...] = jnp.sum(h * ad_ref[...], axis=1, keepdims=True)

    h, es, ed = pl.pallas_call(
        body,
        out_shape=(
            jax.ShapeDtypeStruct((n, _D), jnp.float32),
            jax.ShapeDtypeStruct((n, 1), jnp.float32),
            jax.ShapeDtypeStruct((n, 1), jnp.float32),
        ),
    )(xa, W, a_s.reshape(1, _D), a_d.reshape(1, _D))
    return h, es[:, 0], ed[:, 0]


def _bias_relu_pool(agg, b, w):
    n = agg.shape[0]

    def body(a_ref, b_ref, w_ref, o_ref):
        xf = jnp.maximum(a_ref[...] + b_ref[...], 0.0)
        wv = w_ref[...]
        nrm = jnp.sqrt(jnp.sum(wv * wv))
        score = jnp.sum(xf * wv, axis=1, keepdims=True) / jnp.maximum(nrm, 1e-12)
        o_ref[...] = xf * jnp.tanh(score)

    return pl.pallas_call(
        body, out_shape=jax.ShapeDtypeStruct((n, _D), jnp.float32),
    )(agg, b.reshape(1, _D), w.reshape(1, _D))


def _lin1(x1, x2, x3, W, b):
    n = x1.shape[0]

    def body(a_ref, b2_ref, c_ref, w_ref, bias_ref, o_ref):
        s = a_ref[...] + b2_ref[...] + c_ref[...]
        o_ref[...] = jnp.maximum(
            jnp.dot(s, w_ref[...], preferred_element_type=jnp.float32)
            + bias_ref[...], 0.0)

    return pl.pallas_call(
        body, out_shape=jax.ShapeDtypeStruct((n, _D), jnp.float32),
    )(x1, x2, x3, W, b.reshape(1, _D))


def _mm(x, W):
    n = x.shape[0]

    def body(x_ref, w_ref, o_ref):
        o_ref[...] = jnp.dot(x_ref[...], w_ref[...],
                             preferred_element_type=jnp.float32)

    return pl.pallas_call(
        body, out_shape=jax.ShapeDtypeStruct((n, _D), jnp.float32),
    )(x, W)


def _bias_relu(agg, b):
    n = agg.shape[0]

    def body(a_ref, b_ref, o_ref):
        o_ref[...] = jnp.maximum(a_ref[...] + b_ref[...], 0.0)

    return pl.pallas_call(
        body, out_shape=jax.ShapeDtypeStruct((n, _D), jnp.float32),
    )(agg, b.reshape(1, _D))


def _seg_sum(d, i, n, srt=False):
    return jax.ops.segment_sum(d, i, num_segments=n, indices_are_sorted=srt)


def _seg_max(d, i, n, srt=False):
    return jax.ops.segment_max(d, i, num_segments=n, indices_are_sorted=srt)


def _gat_layer(xa, src, dst, W, a_s, a_d, b, pw):
    n = xa.shape[0]
    h, es, ed = _gat_dense(xa, W, a_s, a_d)
    e = jax.nn.leaky_relu(es[src] + ed[dst], 0.2)
    m = _seg_max(e, dst, n, srt=True)
    m = jnp.where(jnp.isfinite(m), m, 0.0)
    ex = jnp.exp(e - m[dst])
    den = _seg_sum(ex, dst, n, srt=True)
    agg = _seg_sum(ex[:, None] * h[src], dst, n, srt=True)
    agg = agg / jnp.maximum(den, 1e-16)[:, None]
    return _bias_relu_pool(agg, b, pw)


def _gmp_gap(xf, batch, n_graphs):
    cnt = _seg_sum(jnp.ones(xf.shape[0], jnp.float32), batch, n_graphs, srt=True)
    mx = _seg_max(xf, batch, n_graphs, srt=True)
    mx = jnp.where((cnt > 0)[:, None], mx, 0.0)
    mean = _seg_sum(xf, batch, n_graphs, srt=True) / jnp.maximum(cnt, 1.0)[:, None]
    return jnp.concatenate([mx, mean], axis=1)


def kernel(graph_x, graph_edge_index, graph_batch, x, drugEdges, seEdges,
           drugNodes, seNodes, proteinNodes, proteinWeight, nDrug, emb,
           Wg1, as1, ad1, bg1, Wg2, as2, ad2, bg2, Wg3, as3, ad3, bg3,
           pw1, pw2, pw3, lin1_W, lin1_b, Wd1, bd1, Wd2, bd2):
    n = graph_x.shape[0]
    n_graphs = 3000
    loop = jnp.arange(n, dtype=graph_edge_index.dtype)
    src = jnp.concatenate([graph_edge_index[0], loop])
    dst = jnp.concatenate([graph_edge_index[1], loop])
    order = jnp.argsort(dst)
    src, dst = src[order], dst[order]

    xa = _gather_rows(emb, graph_x)
    xa = _gat_layer(xa, src, dst, Wg1, as1, ad1, bg1, pw1)
    x1 = _gmp_gap(xa, graph_batch, n_graphs)
    xa = _gat_layer(xa, src, dst, Wg2, as2, ad2, bg2, pw2)
    x2 = _gmp_gap(xa, graph_batch, n_graphs)
    xa = _gat_layer(xa, src, dst, Wg3, as3, ad3, bg3, pw3)
    x3 = _gmp_gap(xa, graph_batch, n_graphs)
    xdp = _lin1(x1, x2, x3, lin1_W, lin1_b)

    nProtein = proteinNodes.shape[0]
    n_se = x.shape[0] - drugNodes.shape[0] - nProtein
    se_idx = lax.dynamic_slice_in_dim(x, nDrug + nProtein, n_se)
    se = _gather_rows(emb, se_idx)
    X = jnp.concatenate([xdp, se], axis=0)

    nd = X.shape[0]
    dloop = jnp.arange(nd, dtype=drugEdges.dtype)
    dsrc = jnp.concatenate([drugEdges[0], dloop])
    ddst = jnp.concatenate([drugEdges[1], dloop])
    dorder = jnp.argsort(ddst)
    dsrc, ddst = dsrc[dorder], ddst[dorder]
    deg = _seg_sum(jnp.ones(dsrc.shape[0], jnp.float32), ddst, nd, srt=True)
    dis = jnp.where(deg > 0, lax.rsqrt(jnp.maximum(deg, 1e-12)), 0.0)
    norm = dis[dsrc] * dis[ddst]

    h1 = _mm(X, Wd1)
    X = _bias_relu(_seg_sum(norm[:, None] * h1[dsrc], ddst, nd, srt=True), bd1)
    h2 = _mm(X, Wd2)
    X = _bias_relu(_seg_sum(norm[:, None] * h2[dsrc], ddst, nd, srt=True), bd2)

    return (_gather_rows(X, drugNodes), _gather_rows(X, seNodes), X)
